# Initial kernel scaffold; baseline (speedup 1.0000x reference)
import jax
import jax.numpy as jnp
from jax.experimental import pallas as pl


def kernel(traj_pos, traj_theta, map_token_sample_pt):
    raise NotImplementedError("write your pallas kernel here")


# speedup vs baseline: 1.2028x; 1.2028x over previous
import functools

import jax
import jax.numpy as jnp
from jax import lax
from jax.experimental import pallas as pl

N = 16384
K = 2048
BN = 1024
NB = N // BN


def _body(p_ref, th_ref, c_ref, idx_ref, md_ref):
    p = p_ref[...]
    th = th_ref[...]
    cos = jnp.cos(th)
    sin = jnp.sin(th)
    dx1 = p[:, 2:3] - p[:, 0:1]
    dy1 = p[:, 3:4] - p[:, 1:2]
    dx2 = p[:, 4:5] - p[:, 0:1]
    dy2 = p[:, 5:6] - p[:, 1:2]
    px1 = dx1 * cos + dy1 * sin
    py1 = dy1 * cos - dx1 * sin
    px2 = dx2 * cos + dy2 * sin
    py2 = dy2 * cos - dx2 * sin
    pn = dx1 * dx1 + dy1 * dy1 + dx2 * dx2 + dy2 * dy2

    c = c_ref[...]
    cx1 = c[2:3, :]
    cy1 = c[3:4, :]
    cx2 = c[4:5, :]
    cy2 = c[5:6, :]
    e = (c[0:1, :] * c[0:1, :] + c[1:2, :] * c[1:2, :]
         + cx1 * cx1 + cy1 * cy1 + cx2 * cx2 + cy2 * cy2)

    d = e - 2.0 * (px1 * cx1 + py1 * cy1 + px2 * cx2 + py2 * cy2)
    m = jnp.min(d, axis=1, keepdims=True)
    iota = lax.broadcasted_iota(jnp.int32, (BN, K), 1)
    idx = jnp.min(jnp.where(d <= m, iota, K), axis=1)
    idx_ref[...] = idx.reshape(1, 1, BN)
    md_ref[...] = (m[:, 0] + pn[:, 0]).reshape(1, 1, BN)


@jax.jit
def kernel(traj_pos, traj_theta, map_token_sample_pt):
    p = traj_pos.reshape(N, 6)
    th = traj_theta.reshape(N, 1)
    c = map_token_sample_pt.reshape(K, 6).T

    idx3, md3 = pl.pallas_call(
        _body,
        grid=(NB,),
        in_specs=[
            pl.BlockSpec((BN, 6), lambda i: (i, 0)),
            pl.BlockSpec((BN, 1), lambda i: (i, 0)),
            pl.BlockSpec((6, K), lambda i: (0, 0)),
        ],
        out_specs=[
            pl.BlockSpec((1, 1, BN), lambda i: (i, 0, 0)),
            pl.BlockSpec((1, 1, BN), lambda i: (i, 0, 0)),
        ],
        out_shape=[
            jax.ShapeDtypeStruct((NB, 1, BN), jnp.int32),
            jax.ShapeDtypeStruct((NB, 1, BN), jnp.float32),
        ],
    )(p, th, c)

    position = traj_pos[:, 0]
    orientation = traj_theta
    return (position, orientation, idx3.reshape(N), md3.reshape(N))
